# baseline (speedup 1.0000x reference)
import functools

import jax
import jax.numpy as jnp
from jax import lax
from jax.experimental import pallas as pl
from jax.experimental.pallas import tpu as pltpu
from jax.experimental.pallas import tpu_sc as plsc

BUFFER_SIZE = 1000000
OBS_DIM = 64
ACT_DIM = 16
BATCH = 4096

_NUM_CORES = 2
_NUM_SUBCORES = 16
_NW = _NUM_CORES * _NUM_SUBCORES
_BPW = BATCH // _NW


def _sample_kernel(obs_hbm, act_hbm, rew_hbm, nobs_hbm, done_hbm, idx_hbm,
                   out_obs, out_act, out_rew, out_nobs, out_done,
                   idx_v, obs_buf, act_row_buf, act_buf, nobs_buf, rew_v,
                   done_v, s0, s1, s2):
    wid = lax.axis_index("s") * _NUM_CORES + lax.axis_index("c")
    base = wid * _BPW
    pltpu.sync_copy(idx_hbm.at[pl.ds(base, _BPW)], idx_v)
    c_rew = pltpu.async_copy(rew_hbm.at[idx_v], rew_v, s1)
    c_done = pltpu.async_copy(done_hbm.at[idx_v], done_v, s2)
    copies = []
    for g in range(_BPW // 16):
        iv = idx_v[pl.ds(16 * g, 16)]
        qv = lax.shift_right_logical(iv, 3)
        for i in range(16):
            r = iv[i]
            j = 16 * g + i
            copies.append(pltpu.async_copy(
                obs_hbm.at[pl.ds(r, 1)], obs_buf.at[pl.ds(j, 1)], s0))
            copies.append(pltpu.async_copy(
                nobs_hbm.at[pl.ds(r, 1)], nobs_buf.at[pl.ds(j, 1)], s0))
            copies.append(pltpu.async_copy(
                act_hbm.at[pl.ds(qv[i], 1)], act_row_buf.at[pl.ds(j, 1)], s0))
    for cp in copies:
        cp.wait()
    for g in range(_BPW // 16):
        iv = idx_v[pl.ds(16 * g, 16)]
        hv = lax.bitwise_and(iv, 7) * ACT_DIM
        for i in range(16):
            j = 16 * g + i
            act_buf[j, pl.ds(0, 16)] = act_row_buf[j, pl.ds(hv[i], 16)]
    pltpu.sync_copy(obs_buf, out_obs.at[pl.ds(base, _BPW)])
    pltpu.sync_copy(nobs_buf, out_nobs.at[pl.ds(base, _BPW)])
    pltpu.sync_copy(act_buf, out_act.at[pl.ds(base, _BPW)])
    c_rew.wait()
    pltpu.sync_copy(rew_v, out_rew.at[pl.ds(base, _BPW)])
    c_done.wait()
    pltpu.sync_copy(done_v, out_done.at[pl.ds(base, _BPW)])


@jax.jit
def _sample(observations, actions, rewards, next_observations, dones, indices):
    mesh = plsc.VectorSubcoreMesh(core_axis_name="c", subcore_axis_name="s")
    k = functools.partial(
        pl.kernel,
        mesh=mesh,
        out_type=[
            jax.ShapeDtypeStruct((BATCH, OBS_DIM), jnp.float32),
            jax.ShapeDtypeStruct((BATCH, ACT_DIM), jnp.float32),
            jax.ShapeDtypeStruct((BATCH,), jnp.float32),
            jax.ShapeDtypeStruct((BATCH, OBS_DIM), jnp.float32),
            jax.ShapeDtypeStruct((BATCH,), jnp.float32),
        ],
        scratch_types=[
            pltpu.VMEM((_BPW,), jnp.int32),
            pltpu.VMEM((_BPW, OBS_DIM), jnp.float32),
            pltpu.VMEM((_BPW, 128), jnp.float32),
            pltpu.VMEM((_BPW, ACT_DIM), jnp.float32),
            pltpu.VMEM((_BPW, OBS_DIM), jnp.float32),
            pltpu.VMEM((_BPW,), jnp.float32),
            pltpu.VMEM((_BPW,), jnp.float32),
            pltpu.SemaphoreType.DMA,
            pltpu.SemaphoreType.DMA,
            pltpu.SemaphoreType.DMA,
        ],
    )(_sample_kernel)
    act_p = actions.reshape(BUFFER_SIZE // 8, 128)
    return k(observations, act_p, rewards, next_observations, dones, indices)


def kernel(observations, actions, rewards, next_observations, dones, indices):
    idx = indices.astype(jnp.int32)
    out = _sample(observations, actions, rewards, next_observations, dones, idx)
    return tuple(out)


# speedup vs baseline: 1.0875x; 1.0875x over previous
import functools

import jax
import jax.numpy as jnp
from jax import lax
from jax.experimental import pallas as pl
from jax.experimental.pallas import tpu as pltpu
from jax.experimental.pallas import tpu_sc as plsc

BUFFER_SIZE = 1000000
OBS_DIM = 64
ACT_DIM = 16
BATCH = 4096

_NUM_CORES = 2
_NUM_SUBCORES = 16
_NW = _NUM_CORES * _NUM_SUBCORES
_BPW = BATCH // _NW


def _sample_kernel(obs_hbm, act_hbm, rew_hbm, nobs_hbm, done_hbm, idx_hbm,
                   out_obs, out_act, out_rew, out_nobs, out_done,
                   idx_v, obs_buf, act_buf, nobs_buf, rew_v, done_v,
                   s0, s1, s2):
    wid = lax.axis_index("s") * _NUM_CORES + lax.axis_index("c")
    base = wid * _BPW
    pltpu.sync_copy(idx_hbm.at[pl.ds(base, _BPW)], idx_v)
    c_rew = pltpu.async_copy(rew_hbm.at[idx_v], rew_v, s1)
    c_done = pltpu.async_copy(done_hbm.at[idx_v], done_v, s2)
    copies = []
    for g in range(_BPW // 16):
        iv = idx_v[pl.ds(16 * g, 16)]
        for i in range(16):
            r = iv[i]
            j = 16 * g + i
            copies.append(pltpu.async_copy(
                obs_hbm.at[pl.ds(r, 1)], obs_buf.at[pl.ds(j, 1)], s0))
            copies.append(pltpu.async_copy(
                nobs_hbm.at[pl.ds(r, 1)], nobs_buf.at[pl.ds(j, 1)], s0))
            copies.append(pltpu.async_copy(
                act_hbm.at[pl.ds(r, 1)], act_buf.at[pl.ds(j, 1)], s0))
    for cp in copies:
        cp.wait()
    pltpu.sync_copy(obs_buf, out_obs.at[pl.ds(base, _BPW)])
    pltpu.sync_copy(nobs_buf, out_nobs.at[pl.ds(base, _BPW)])
    pltpu.sync_copy(act_buf, out_act.at[pl.ds(base, _BPW)])
    c_rew.wait()
    pltpu.sync_copy(rew_v, out_rew.at[pl.ds(base, _BPW)])
    c_done.wait()
    pltpu.sync_copy(done_v, out_done.at[pl.ds(base, _BPW)])


@jax.jit
def _sample(observations, actions, rewards, next_observations, dones, indices):
    mesh = plsc.VectorSubcoreMesh(core_axis_name="c", subcore_axis_name="s")
    k = functools.partial(
        pl.kernel,
        mesh=mesh,
        out_type=[
            jax.ShapeDtypeStruct((BATCH, OBS_DIM), jnp.float32),
            jax.ShapeDtypeStruct((BATCH, ACT_DIM), jnp.float32),
            jax.ShapeDtypeStruct((BATCH,), jnp.float32),
            jax.ShapeDtypeStruct((BATCH, OBS_DIM), jnp.float32),
            jax.ShapeDtypeStruct((BATCH,), jnp.float32),
        ],
        scratch_types=[
            pltpu.VMEM((_BPW,), jnp.int32),
            pltpu.VMEM((_BPW, OBS_DIM), jnp.float32),
            pltpu.VMEM((_BPW, ACT_DIM), jnp.float32),
            pltpu.VMEM((_BPW, OBS_DIM), jnp.float32),
            pltpu.VMEM((_BPW,), jnp.float32),
            pltpu.VMEM((_BPW,), jnp.float32),
            pltpu.SemaphoreType.DMA,
            pltpu.SemaphoreType.DMA,
            pltpu.SemaphoreType.DMA,
        ],
    )(_sample_kernel)
    return k(observations, actions, rewards, next_observations, dones, indices)


def kernel(observations, actions, rewards, next_observations, dones, indices):
    idx = indices.astype(jnp.int32)
    out = _sample(observations, actions, rewards, next_observations, dones, idx)
    return tuple(out)
